# Initial kernel scaffold; baseline (speedup 1.0000x reference)
import jax
import jax.numpy as jnp
from jax.experimental import pallas as pl


def kernel(x, edge_index, edge_attr, W1, b1, W2, b2, emb1, emb2):
    raise NotImplementedError("write your pallas kernel here")


# speedup vs baseline: 2.0050x; 2.0050x over previous
import functools

import jax
import jax.numpy as jnp
from jax import lax
from jax.experimental import pallas as pl
from jax.experimental.pallas import tpu as pltpu
from jax.experimental.pallas import tpu_sc as plsc

N = 10000
NPAD = 10240
E = 320000
D = 128
NC = 2
NS = 16
NW = NC * NS
EPW = E // NW
CHUNK = 80
NCHUNK = EPW // CHUNK
RPW = NPAD // NS


def _mtab_body(e1_ref, e2_ref, m_ref):
    ct_i = lax.broadcasted_iota(jnp.int32, (16, 8), 0)
    sel_i = lax.broadcasted_iota(jnp.int32, (16, 8), 1)
    valid = ct_i < 15
    s1 = (((ct_i // 3) == sel_i) & valid).astype(jnp.float32)
    s2 = (((ct_i % 3) == sel_i) & valid).astype(jnp.float32)
    m_ref[...] = (jnp.dot(s1, e1_ref[...], preferred_element_type=jnp.float32)
                  + jnp.dot(s2, e2_ref[...], preferred_element_type=jnp.float32))


def _sc_body(x_hbm, m_hbm, eidx_hbm, zx_hbm, px_hbm,
             acc_x, idx_v, ct_v, rows_v, mrows_v, sem, sem2):
    c = lax.axis_index("c")
    s = lax.axis_index("s")
    wid = c * NS + s

    pltpu.sync_copy(zx_hbm, acc_x.at[pl.ds(s * RPW, RPW)])
    plsc.subcore_barrier()

    def chunk_body(g, carry):
        pltpu.sync_copy(eidx_hbm.at[wid, g], idx_v)
        cpx = pltpu.async_copy(x_hbm.at[idx_v.at[0]], rows_v, sem)
        for k in range(CHUNK // 16):
            sl = pl.ds(k * 16, 16)
            ct_v[0, sl] = idx_v[2, sl] * 3 + idx_v[3, sl]
        cpm = pltpu.async_copy(m_hbm.at[ct_v.at[0]], mrows_v, sem2)
        cpx.wait()
        pltpu.sync_copy(rows_v, acc_x.at[idx_v.at[1]], add=True)
        cpm.wait()
        pltpu.sync_copy(mrows_v, acc_x.at[idx_v.at[1]], add=True)
        return carry

    lax.fori_loop(0, NCHUNK, chunk_body, 0)
    plsc.subcore_barrier()

    pltpu.sync_copy(acc_x.at[pl.ds(s * RPW, RPW)], px_hbm.at[c, pl.ds(s * RPW, RPW)])


_sc_kernel = functools.partial(
    pl.kernel,
    out_type=jax.ShapeDtypeStruct((NC, NPAD, D), jnp.float32),
    mesh=plsc.VectorSubcoreMesh(core_axis_name="c", subcore_axis_name="s",
                                num_cores=NC, num_subcores=NS),
    scratch_types=[
        pltpu.VMEM_SHARED((NPAD, D), jnp.float32),
        pltpu.VMEM((4, CHUNK), jnp.int32),
        pltpu.VMEM((1, CHUNK), jnp.int32),
        pltpu.VMEM((CHUNK, D), jnp.float32),
        pltpu.VMEM((CHUNK, D), jnp.float32),
        pltpu.SemaphoreType.DMA,
        pltpu.SemaphoreType.DMA,
    ],
)(_sc_body)


def _mlp_body(px_ref, x_ref, w1_ref, b1_ref, w2_ref, b2_ref, o_ref):
    out = px_ref[0] + px_ref[1] + x_ref[...]
    h = jnp.maximum(jnp.dot(out, w1_ref[...],
                            preferred_element_type=jnp.float32) + b1_ref[...], 0.0)
    o_ref[...] = jnp.dot(h, w2_ref[...],
                         preferred_element_type=jnp.float32) + b2_ref[...]


def kernel(x, edge_index, edge_attr, W1, b1, W2, b2, emb1, emb2):
    ei = edge_index.astype(jnp.int32)
    ea = edge_attr.astype(jnp.int32)
    eidx = jnp.stack(
        [ei[1].reshape(NW, NCHUNK, CHUNK), ei[0].reshape(NW, NCHUNK, CHUNK),
         ea[:, 0].reshape(NW, NCHUNK, CHUNK), ea[:, 1].reshape(NW, NCHUNK, CHUNK)],
        axis=2)
    zx = jnp.zeros((RPW, D), jnp.float32)
    e1p = jnp.pad(emb1, ((0, 3), (0, 0)))
    e2p = jnp.pad(emb2, ((0, 5), (0, 0)))

    m = pl.pallas_call(
        _mtab_body,
        out_shape=jax.ShapeDtypeStruct((16, D), jnp.float32),
    )(e1p, e2p)

    px = _sc_kernel(x, m, eidx, zx)

    bn = 2000
    grid = (N // bn,)
    out = pl.pallas_call(
        _mlp_body,
        grid=grid,
        in_specs=[
            pl.BlockSpec((NC, bn, D), lambda i: (0, i, 0)),
            pl.BlockSpec((bn, D), lambda i: (i, 0)),
            pl.BlockSpec((D, 2 * D), lambda i: (0, 0)),
            pl.BlockSpec((1, 2 * D), lambda i: (0, 0)),
            pl.BlockSpec((2 * D, D), lambda i: (0, 0)),
            pl.BlockSpec((1, D), lambda i: (0, 0)),
        ],
        out_specs=pl.BlockSpec((bn, D), lambda i: (i, 0)),
        out_shape=jax.ShapeDtypeStruct((N, D), jnp.float32),
    )(px, x, W1, b1.reshape(1, -1), W2, b2.reshape(1, -1))
    return out


# speedup vs baseline: 2.0236x; 1.0093x over previous
import functools

import jax
import jax.numpy as jnp
from jax import lax
from jax.experimental import pallas as pl
from jax.experimental.pallas import tpu as pltpu
from jax.experimental.pallas import tpu_sc as plsc

N = 10000
NPAD = 10240
E = 320000
D = 128
NC = 2
NS = 16
NW = NC * NS
CPW = 2 * E // NW
CHUNK = 80
NCH = CPW // CHUNK
RPW = NPAD // NS
NBUF = 3
NIBUF = 6


def _mtab_body(e1_ref, e2_ref, m_ref):
    ct_i = lax.broadcasted_iota(jnp.int32, (16, 8), 0)
    sel_i = lax.broadcasted_iota(jnp.int32, (16, 8), 1)
    valid = ct_i < 15
    s1 = (((ct_i // 3) == sel_i) & valid).astype(jnp.float32)
    s2 = (((ct_i % 3) == sel_i) & valid).astype(jnp.float32)
    m_ref[...] = (jnp.dot(s1, e1_ref[...], preferred_element_type=jnp.float32)
                  + jnp.dot(s2, e2_ref[...], preferred_element_type=jnp.float32))


def _sc_body(xm_hbm, eidx_hbm, zx_hbm, px_hbm,
             acc_x, idx_v, rows_v, isem, gsem, ssem):
    c = lax.axis_index("c")
    s = lax.axis_index("s")
    wid = c * NS + s

    pltpu.sync_copy(zx_hbm, acc_x.at[pl.ds(s * RPW, RPW)])
    plsc.subcore_barrier()

    def wait_bytes(dst, sem):
        pltpu.make_async_copy(xm_hbm.at[pl.ds(0, CHUNK)], dst, sem).wait()

    def load_idx(g):
        pltpu.async_copy(eidx_hbm.at[wid, g], idx_v.at[g % NIBUF],
                         isem.at[g % NIBUF])

    def start_gather(g):
        b = g % NBUF
        pltpu.async_copy(xm_hbm.at[idx_v.at[g % NIBUF, 0]], rows_v.at[b],
                         gsem.at[b])

    for g in range(4):
        load_idx(g)
    pltpu.make_async_copy(eidx_hbm.at[wid, 0], idx_v.at[0], isem.at[0]).wait()
    start_gather(0)
    pltpu.make_async_copy(eidx_hbm.at[wid, 1], idx_v.at[1], isem.at[1]).wait()
    start_gather(1)

    def chunk_body(g, carry):
        b = g % NBUF
        wait_bytes(rows_v.at[b], gsem.at[b])
        pltpu.async_copy(rows_v.at[b], acc_x.at[idx_v.at[g % NIBUF, 1]],
                         ssem.at[b], add=True)

        @pl.when(g >= 1)
        def _():
            wait_bytes(rows_v.at[(g - 1) % NBUF], ssem.at[(g - 1) % NBUF])

        @pl.when(g + 4 < NCH)
        def _():
            load_idx(g + 4)

        @pl.when(g + 2 < NCH)
        def _():
            b2 = (g + 2) % NBUF
            pltpu.make_async_copy(eidx_hbm.at[wid, 0],
                                  idx_v.at[(g + 2) % NIBUF],
                                  isem.at[(g + 2) % NIBUF]).wait()
            start_gather(g + 2)

        return carry

    lax.fori_loop(0, NCH, chunk_body, 0)
    wait_bytes(rows_v.at[(NCH - 1) % NBUF], ssem.at[(NCH - 1) % NBUF])
    plsc.subcore_barrier()

    pltpu.sync_copy(acc_x.at[pl.ds(s * RPW, RPW)], px_hbm.at[c, pl.ds(s * RPW, RPW)])


_sc_kernel = functools.partial(
    pl.kernel,
    out_type=jax.ShapeDtypeStruct((NC, NPAD, D), jnp.float32),
    mesh=plsc.VectorSubcoreMesh(core_axis_name="c", subcore_axis_name="s",
                                num_cores=NC, num_subcores=NS),
    scratch_types=[
        pltpu.VMEM_SHARED((NPAD, D), jnp.float32),
        pltpu.VMEM((NIBUF, 2, CHUNK), jnp.int32),
        pltpu.VMEM((NBUF, CHUNK, D), jnp.float32),
        pltpu.SemaphoreType.DMA((NIBUF,)),
        pltpu.SemaphoreType.DMA((NBUF,)),
        pltpu.SemaphoreType.DMA((NBUF,)),
    ],
)(_sc_body)


def _mlp_body(px_ref, x_ref, w1_ref, b1_ref, w2_ref, b2_ref, o_ref):
    out = px_ref[0] + px_ref[1] + x_ref[...]
    h = jnp.maximum(jnp.dot(out, w1_ref[...],
                            preferred_element_type=jnp.float32) + b1_ref[...], 0.0)
    o_ref[...] = jnp.dot(h, w2_ref[...],
                         preferred_element_type=jnp.float32) + b2_ref[...]


def kernel(x, edge_index, edge_attr, W1, b1, W2, b2, emb1, emb2):
    ei = edge_index.astype(jnp.int32)
    ea = edge_attr.astype(jnp.int32)
    row = ei[0].reshape(NW, E // NW)
    col = ei[1].reshape(NW, E // NW)
    ct = (ea[:, 0] * 3 + ea[:, 1] + N).reshape(NW, E // NW)
    src = jnp.concatenate([col, ct], axis=1).reshape(NW, NCH, CHUNK)
    dst = jnp.concatenate([row, row], axis=1).reshape(NW, NCH, CHUNK)
    eidx = jnp.stack([src, dst], axis=2)
    zx = jnp.zeros((RPW, D), jnp.float32)
    e1p = jnp.pad(emb1, ((0, 3), (0, 0)))
    e2p = jnp.pad(emb2, ((0, 5), (0, 0)))

    m = pl.pallas_call(
        _mtab_body,
        out_shape=jax.ShapeDtypeStruct((16, D), jnp.float32),
    )(e1p, e2p)
    xm = jnp.concatenate([x, m], axis=0)

    px = _sc_kernel(xm, eidx, zx)

    bn = 2000
    grid = (N // bn,)
    out = pl.pallas_call(
        _mlp_body,
        grid=grid,
        in_specs=[
            pl.BlockSpec((NC, bn, D), lambda i: (0, i, 0)),
            pl.BlockSpec((bn, D), lambda i: (i, 0)),
            pl.BlockSpec((D, 2 * D), lambda i: (0, 0)),
            pl.BlockSpec((1, 2 * D), lambda i: (0, 0)),
            pl.BlockSpec((2 * D, D), lambda i: (0, 0)),
            pl.BlockSpec((1, D), lambda i: (0, 0)),
        ],
        out_specs=pl.BlockSpec((bn, D), lambda i: (i, 0)),
        out_shape=jax.ShapeDtypeStruct((N, D), jnp.float32),
    )(px, x, W1, b1.reshape(1, -1), W2, b2.reshape(1, -1))
    return out


# speedup vs baseline: 3.8630x; 1.9090x over previous
import functools

import jax
import jax.numpy as jnp
from jax import lax
from jax.experimental import pallas as pl
from jax.experimental.pallas import tpu as pltpu
from jax.experimental.pallas import tpu_sc as plsc

N = 10000
NPAD = 10240
E = 320000
D = 128
NC = 2
NS = 16
NW = NC * NS
EPW = E // NW
CHUNK = 80
NCHUNK = EPW // CHUNK
RPW = NPAD // NS


def _mtab_body(e1_ref, e2_ref, m_ref):
    ct_i = lax.broadcasted_iota(jnp.int32, (16, 8), 0)
    sel_i = lax.broadcasted_iota(jnp.int32, (16, 8), 1)
    valid = ct_i < 15
    s1 = (((ct_i // 3) == sel_i) & valid).astype(jnp.float32)
    s2 = (((ct_i % 3) == sel_i) & valid).astype(jnp.float32)
    m_ref[...] = (jnp.dot(s1, e1_ref[...], preferred_element_type=jnp.float32)
                  + jnp.dot(s2, e2_ref[...], preferred_element_type=jnp.float32))


def _sc_body(x_hbm, m_hbm, eidx_hbm, zx_hbm, px_hbm,
             acc_x, m_tab, idx_v, rows_v, sem):
    c = lax.axis_index("c")
    s = lax.axis_index("s")
    wid = c * NS + s

    pltpu.sync_copy(zx_hbm, acc_x.at[pl.ds(s * RPW, RPW)])
    pltpu.sync_copy(m_hbm, m_tab)
    plsc.subcore_barrier()

    def chunk_body(g, carry):
        pltpu.sync_copy(eidx_hbm.at[wid, g], idx_v)
        cpx = pltpu.async_copy(x_hbm.at[idx_v.at[0]], rows_v, sem)
        cpx.wait()
        for k in range(CHUNK // 16):
            sl = pl.ds(k * 16, 16)
            ctv = idx_v[2, sl] * 3 + idx_v[3, sl]
            for l in range(16):
                r = k * 16 + l
                ct = ctv[l]
                for v in range(D // 16):
                    vsl = pl.ds(v * 16, 16)
                    rows_v[r, vsl] = rows_v[r, vsl] + m_tab[ct, vsl]
        pltpu.sync_copy(rows_v, acc_x.at[idx_v.at[1]], add=True)
        return carry

    lax.fori_loop(0, NCHUNK, chunk_body, 0)
    plsc.subcore_barrier()

    pltpu.sync_copy(acc_x.at[pl.ds(s * RPW, RPW)], px_hbm.at[c, pl.ds(s * RPW, RPW)])


_sc_kernel = functools.partial(
    pl.kernel,
    out_type=jax.ShapeDtypeStruct((NC, NPAD, D), jnp.float32),
    mesh=plsc.VectorSubcoreMesh(core_axis_name="c", subcore_axis_name="s",
                                num_cores=NC, num_subcores=NS),
    scratch_types=[
        pltpu.VMEM_SHARED((NPAD, D), jnp.float32),
        pltpu.VMEM((16, D), jnp.float32),
        pltpu.VMEM((4, CHUNK), jnp.int32),
        pltpu.VMEM((CHUNK, D), jnp.float32),
        pltpu.SemaphoreType.DMA,
    ],
)(_sc_body)


def _mlp_body(px_ref, x_ref, w1_ref, b1_ref, w2_ref, b2_ref, o_ref):
    out = px_ref[0] + px_ref[1] + x_ref[...]
    h = jnp.maximum(jnp.dot(out, w1_ref[...],
                            preferred_element_type=jnp.float32) + b1_ref[...], 0.0)
    o_ref[...] = jnp.dot(h, w2_ref[...],
                         preferred_element_type=jnp.float32) + b2_ref[...]


def kernel(x, edge_index, edge_attr, W1, b1, W2, b2, emb1, emb2):
    ei = edge_index.astype(jnp.int32)
    ea = edge_attr.astype(jnp.int32)
    eidx = jnp.stack(
        [ei[1].reshape(NW, NCHUNK, CHUNK), ei[0].reshape(NW, NCHUNK, CHUNK),
         ea[:, 0].reshape(NW, NCHUNK, CHUNK), ea[:, 1].reshape(NW, NCHUNK, CHUNK)],
        axis=2)
    zx = jnp.zeros((RPW, D), jnp.float32)
    e1p = jnp.pad(emb1, ((0, 3), (0, 0)))
    e2p = jnp.pad(emb2, ((0, 5), (0, 0)))

    m = pl.pallas_call(
        _mtab_body,
        out_shape=jax.ShapeDtypeStruct((16, D), jnp.float32),
    )(e1p, e2p)

    px = _sc_kernel(x, m, eidx, zx)

    bn = 2000
    grid = (N // bn,)
    out = pl.pallas_call(
        _mlp_body,
        grid=grid,
        in_specs=[
            pl.BlockSpec((NC, bn, D), lambda i: (0, i, 0)),
            pl.BlockSpec((bn, D), lambda i: (i, 0)),
            pl.BlockSpec((D, 2 * D), lambda i: (0, 0)),
            pl.BlockSpec((1, 2 * D), lambda i: (0, 0)),
            pl.BlockSpec((2 * D, D), lambda i: (0, 0)),
            pl.BlockSpec((1, D), lambda i: (0, 0)),
        ],
        out_specs=pl.BlockSpec((bn, D), lambda i: (i, 0)),
        out_shape=jax.ShapeDtypeStruct((N, D), jnp.float32),
    )(px, x, W1, b1.reshape(1, -1), W2, b2.reshape(1, -1))
    return out


# speedup vs baseline: 4.0571x; 1.0502x over previous
import functools

import jax
import jax.numpy as jnp
from jax import lax
from jax.experimental import pallas as pl
from jax.experimental.pallas import tpu as pltpu
from jax.experimental.pallas import tpu_sc as plsc

N = 10000
NPAD = 10240
E = 320000
D = 128
NC = 2
NS = 16
NW = NC * NS
EPW = E // NW
CHUNK = 80
NCHUNK = EPW // CHUNK
RPW = NPAD // NS


def _mtab_body(e1_ref, e2_ref, m_ref):
    ct_i = lax.broadcasted_iota(jnp.int32, (16, 8), 0)
    sel_i = lax.broadcasted_iota(jnp.int32, (16, 8), 1)
    valid = ct_i < 15
    s1 = (((ct_i // 3) == sel_i) & valid).astype(jnp.float32)
    s2 = (((ct_i % 3) == sel_i) & valid).astype(jnp.float32)
    m_ref[...] = (jnp.dot(s1, e1_ref[...], preferred_element_type=jnp.float32)
                  + jnp.dot(s2, e2_ref[...], preferred_element_type=jnp.float32))


def _sc_body(x_hbm, m_hbm, eidx_hbm, zx_hbm, px_hbm,
             acc_x, m_tab, idx_v, rows_v, sem0, sem1):
    c = lax.axis_index("c")
    s = lax.axis_index("s")
    wid = c * NS + s
    sems = (sem0, sem1)

    pltpu.sync_copy(zx_hbm, acc_x.at[pl.ds(s * RPW, RPW)])
    pltpu.sync_copy(m_hbm, m_tab)
    plsc.subcore_barrier()

    def process(g, a):
        b = 1 - a
        pltpu.sync_copy(eidx_hbm.at[wid, g + 1], idx_v.at[b])
        pltpu.async_copy(x_hbm.at[idx_v.at[b, 0]], rows_v.at[b], sems[b])
        pltpu.make_async_copy(x_hbm.at[pl.ds(0, CHUNK)], rows_v.at[a],
                              sems[a]).wait()
        for k in range(CHUNK // 16):
            sl = pl.ds(k * 16, 16)
            ctv = idx_v[a, 2, sl] * 3 + idx_v[a, 3, sl]
            for l in range(16):
                r = k * 16 + l
                ct = ctv[l]
                for v in range(D // 16):
                    vsl = pl.ds(v * 16, 16)
                    rows_v[a, r, vsl] = rows_v[a, r, vsl] + m_tab[ct, vsl]
        pltpu.sync_copy(rows_v.at[a], acc_x.at[idx_v.at[a, 1]], add=True)

    pltpu.sync_copy(eidx_hbm.at[wid, 0], idx_v.at[0])
    pltpu.async_copy(x_hbm.at[idx_v.at[0, 0]], rows_v.at[0], sems[0])
    process(0, 0)

    def pair(i, carry):
        process(1 + 2 * i, 1)
        process(2 + 2 * i, 0)
        return carry

    lax.fori_loop(0, (NCHUNK - 1) // 2, pair, 0)

    pltpu.make_async_copy(x_hbm.at[pl.ds(0, CHUNK)], rows_v.at[1],
                          sems[1]).wait()
    plsc.subcore_barrier()

    pltpu.sync_copy(acc_x.at[pl.ds(s * RPW, RPW)], px_hbm.at[c, pl.ds(s * RPW, RPW)])


_sc_kernel = functools.partial(
    pl.kernel,
    out_type=jax.ShapeDtypeStruct((NC, NPAD, D), jnp.float32),
    mesh=plsc.VectorSubcoreMesh(core_axis_name="c", subcore_axis_name="s",
                                num_cores=NC, num_subcores=NS),
    scratch_types=[
        pltpu.VMEM_SHARED((NPAD, D), jnp.float32),
        pltpu.VMEM((16, D), jnp.float32),
        pltpu.VMEM((2, 4, CHUNK), jnp.int32),
        pltpu.VMEM((2, CHUNK, D), jnp.float32),
        pltpu.SemaphoreType.DMA,
        pltpu.SemaphoreType.DMA,
    ],
)(_sc_body)


def _mlp_body(px_ref, x_ref, w1_ref, b1_ref, w2_ref, b2_ref, o_ref):
    out = px_ref[0] + px_ref[1] + x_ref[...]
    h = jnp.maximum(jnp.dot(out, w1_ref[...],
                            preferred_element_type=jnp.float32) + b1_ref[...], 0.0)
    o_ref[...] = jnp.dot(h, w2_ref[...],
                         preferred_element_type=jnp.float32) + b2_ref[...]


def kernel(x, edge_index, edge_attr, W1, b1, W2, b2, emb1, emb2):
    ei = edge_index.astype(jnp.int32)
    ea = edge_attr.astype(jnp.int32)
    eidx = jnp.stack(
        [ei[1].reshape(NW, NCHUNK, CHUNK), ei[0].reshape(NW, NCHUNK, CHUNK),
         ea[:, 0].reshape(NW, NCHUNK, CHUNK), ea[:, 1].reshape(NW, NCHUNK, CHUNK)],
        axis=2)
    eidx = jnp.pad(eidx, ((0, 0), (0, 1), (0, 0), (0, 0)))
    zx = jnp.zeros((RPW, D), jnp.float32)
    e1p = jnp.pad(emb1, ((0, 3), (0, 0)))
    e2p = jnp.pad(emb2, ((0, 5), (0, 0)))

    m = pl.pallas_call(
        _mtab_body,
        out_shape=jax.ShapeDtypeStruct((16, D), jnp.float32),
    )(e1p, e2p)

    px = _sc_kernel(x, m, eidx, zx)

    bn = 2000
    grid = (N // bn,)
    out = pl.pallas_call(
        _mlp_body,
        grid=grid,
        in_specs=[
            pl.BlockSpec((NC, bn, D), lambda i: (0, i, 0)),
            pl.BlockSpec((bn, D), lambda i: (i, 0)),
            pl.BlockSpec((D, 2 * D), lambda i: (0, 0)),
            pl.BlockSpec((1, 2 * D), lambda i: (0, 0)),
            pl.BlockSpec((2 * D, D), lambda i: (0, 0)),
            pl.BlockSpec((1, D), lambda i: (0, 0)),
        ],
        out_specs=pl.BlockSpec((bn, D), lambda i: (i, 0)),
        out_shape=jax.ShapeDtypeStruct((N, D), jnp.float32),
    )(px, x, W1, b1.reshape(1, -1), W2, b2.reshape(1, -1))
    return out
